# baseline (speedup 1.0000x reference)
import functools

import jax
import jax.numpy as jnp
from jax import lax
from jax.experimental import pallas as pl
from jax.experimental.pallas import tpu as pltpu
from jax.experimental.pallas import tpu_sc as plsc

NC, NS, LANES = 2, 16, 16
NW = NC * NS


def _bcast_lane(v, j):
    idx = jnp.full((LANES, 1), j, jnp.int32)
    dnums = lax.GatherDimensionNumbers(
        offset_dims=(), collapsed_slice_dims=(0,), start_index_map=(0,))
    return lax.gather(v, idx, dnums, (1,),
                      mode=lax.GatherScatterMode.PROMISE_IN_BOUNDS)


def _mesh():
    return plsc.VectorSubcoreMesh(core_axis_name="c", subcore_axis_name="s",
                                  num_cores=NC, num_subcores=NS)


def _sc_embed_gather(table, idx3, D):
    nw, nchunks, chunk = idx3.shape
    total = nw * nchunks * chunk
    n_per_w = nchunks * chunk

    @functools.partial(
        pl.kernel,
        out_type=jax.ShapeDtypeStruct((total, D), jnp.float32),
        mesh=_mesh(),
        compiler_params=pltpu.CompilerParams(needs_layout_passes=False),
        scratch_types=[
            pltpu.VMEM((nchunks, chunk), jnp.int32),
            pltpu.VMEM((2, chunk, D), jnp.float32),
            pltpu.SemaphoreType.DMA,
        ],
    )
    def k(table_hbm, idx_hbm, out_hbm, idx_v, rows_v, gsem):
        wid = lax.axis_index("s") * NC + lax.axis_index("c")
        base = wid * n_per_w
        pltpu.sync_copy(idx_hbm.at[wid], idx_v)
        pltpu.async_copy(table_hbm.at[idx_v.at[0]], rows_v.at[0], gsem)

        def body(i, _):
            buf = lax.rem(i, 2)

            @pl.when(i + 1 < nchunks)
            def _start():
                pltpu.async_copy(table_hbm.at[idx_v.at[i + 1]],
                                 rows_v.at[1 - buf], gsem)

            pltpu.make_async_copy(table_hbm.at[idx_v.at[i]],
                                  rows_v.at[buf], gsem).wait()
            pltpu.sync_copy(
                rows_v.at[buf],
                out_hbm.at[pl.ds(pl.multiple_of(base + i * chunk, 8), chunk)])
            return 0

        lax.fori_loop(0, nchunks, body, 0)

    return k(table, idx3)


def _sc_edge_aggregate(h_all_flat, eidx_r, dst_r, norm_r, cnt2, base2,
                       half, dout, chunk, ecap_r):
    acc_rows = half + 8
    zrows = -(-(-(-acc_rows // NS)) // 8) * 8
    erows = -(-(-(-half // NS)) // 8) * 8

    def _pieces(total):
        out_p, off = [], 0
        while off < total:
            sz = min(chunk, total - off)
            out_p.append((off, sz))
            off += sz
        return out_p

    jslices = dout // LANES
    groups = chunk // LANES

    @functools.partial(
        pl.kernel,
        out_type=jax.ShapeDtypeStruct((NC, half, dout), jnp.float32),
        mesh=_mesh(),
        compiler_params=pltpu.CompilerParams(needs_layout_passes=False),
        scratch_types=[
            pltpu.VMEM((4, chunk), jnp.int32),
            pltpu.VMEM((4, chunk), jnp.int32),
            pltpu.VMEM((4, chunk), jnp.float32),
            pltpu.VMEM((2, chunk, dout), jnp.float32),
            pltpu.VMEM((2, chunk, dout), jnp.float32),
            pltpu.VMEM((LANES,), jnp.int32),
            pltpu.VMEM((LANES,), jnp.int32),
            pltpu.VMEM_SHARED((acc_rows, dout), jnp.float32),
            pltpu.SemaphoreType.DMA,
            pltpu.SemaphoreType.DMA,
            pltpu.SemaphoreType.DMA,
        ],
    )
    def k(hall_hbm, eidx_hbm, dst_hbm, norm_hbm, cnt_hbm, base_hbm, out_hbm,
          eidx_m, dst_m, norm_m, rows_v, sbuf, cntv, basev, acc,
          gsem, msem, ssem):
        c = lax.axis_index("c")
        s = lax.axis_index("s")
        pltpu.sync_copy(cnt_hbm.at[c], cntv)
        pltpu.sync_copy(base_hbm.at[c], basev)
        lanemask = (lax.iota(jnp.int32, LANES) == s).astype(jnp.int32)
        nck = jnp.sum(cntv[...] * lanemask)
        bw = jnp.sum(basev[...] * lanemask)
        ebase = c * ecap_r + bw

        def meta_start(i, slot):
            off = pl.multiple_of(ebase + i * chunk, 8)
            pltpu.async_copy(eidx_hbm.at[pl.ds(off, chunk)],
                             eidx_m.at[slot], msem)
            pltpu.async_copy(dst_hbm.at[pl.ds(off, chunk)],
                             dst_m.at[slot], msem)
            pltpu.async_copy(norm_hbm.at[pl.ds(off, chunk)],
                             norm_m.at[slot], msem)

        def meta_wait(i, slot):
            off = pl.multiple_of(ebase + i * chunk, 8)
            pltpu.make_async_copy(eidx_hbm.at[pl.ds(off, chunk)],
                                  eidx_m.at[slot], msem).wait()
            pltpu.make_async_copy(dst_hbm.at[pl.ds(off, chunk)],
                                  dst_m.at[slot], msem).wait()
            pltpu.make_async_copy(norm_hbm.at[pl.ds(off, chunk)],
                                  norm_m.at[slot], msem).wait()

        zero16 = jnp.zeros((LANES,), jnp.float32)

        def zstore(r, _):
            for j in range(jslices):
                rows_v[0, r, pl.ds(j * LANES, LANES)] = zero16
            return 0

        lax.fori_loop(0, chunk, zstore, 0)
        zrow0 = pl.multiple_of(jnp.minimum(s * zrows, acc_rows - zrows), 8)
        for (zoff, zsz) in _pieces(zrows):
            pltpu.sync_copy(rows_v.at[0, pl.ds(0, zsz)],
                            acc.at[pl.ds(zrow0 + zoff, zsz)])
        plsc.subcore_barrier()

        @pl.when(nck > 0)
        def _run():
            meta_start(0, 0)
            meta_wait(0, 0)
            pltpu.async_copy(hall_hbm.at[eidx_m.at[0]], rows_v.at[0], gsem)

            @pl.when(1 < nck)
            def _prime():
                meta_start(1, 1)

            def chunk_step(i, buf):
                mb = lax.rem(i, 4)
                m1 = lax.rem(i + 1, 4)
                m2 = lax.rem(i + 2, 4)

                @pl.when(i + 1 < nck)
                def _start():
                    meta_wait(i + 1, m1)
                    pltpu.async_copy(hall_hbm.at[eidx_m.at[m1]],
                                     rows_v.at[1 - buf], gsem)

                @pl.when(i >= 2)
                def _drain():
                    pltpu.make_async_copy(sbuf.at[buf],
                                          acc.at[dst_m.at[m2]], ssem).wait()

                @pl.when(i + 2 < nck)
                def _start2():
                    meta_start(i + 2, m2)

                pltpu.make_async_copy(hall_hbm.at[eidx_m.at[mb]],
                                      rows_v.at[buf], gsem).wait()

                for g in range(groups):
                    nv = norm_m[mb, pl.ds(g * LANES, LANES)]
                    for j in range(LANES):
                        r = g * LANES + j
                        bj = _bcast_lane(nv, j)
                        for jj in range(jslices):
                            sl = pl.ds(jj * LANES, LANES)
                            sbuf[buf, r, sl] = rows_v[buf, r, sl] * bj

                pltpu.async_copy(sbuf.at[buf], acc.at[dst_m.at[mb]], ssem,
                                 add=True)

            def pair(p, _):
                chunk_step(2 * p, 0)
                chunk_step(2 * p + 1, 1)
                return 0

            lax.fori_loop(0, nck // 2, pair, 0)
            pltpu.make_async_copy(sbuf.at[0],
                                  acc.at[dst_m.at[0]], ssem).wait()
            pltpu.make_async_copy(sbuf.at[1],
                                  acc.at[dst_m.at[0]], ssem).wait()

        plsc.subcore_barrier()
        erow0 = pl.multiple_of(jnp.minimum(s * erows, half - erows), 8)
        for (zoff, zsz) in _pieces(erows):
            pltpu.sync_copy(acc.at[pl.ds(erow0 + zoff, zsz)],
                            out_hbm.at[c, pl.ds(erow0 + zoff, zsz)])

    return k(h_all_flat, eidx_r, dst_r, norm_r, cnt2, base2)


def _tc_lstm_fc(x3, wih_t, whh_t, b_row, wfc_t, bfc_row, nb):
    lseq, n_nodes, xw = x3.shape
    h_dim = wih_t.shape[0]
    din = wfc_t.shape[1]

    def body(x_ref, wih_ref, whh_ref, b_ref, wfc_ref, bfc_ref, out_ref,
             h_ref, c_ref):
        h_ref[...] = jnp.zeros_like(h_ref)
        c_ref[...] = jnp.zeros_like(c_ref)

        def step(t, _):
            xt = x_ref[lseq - 1 - t][:, 0:h_dim]
            gates = (jnp.dot(xt, wih_ref[...],
                             preferred_element_type=jnp.float32)
                     + jnp.dot(h_ref[...], whh_ref[...],
                               preferred_element_type=jnp.float32)
                     + b_ref[...])
            gi = jax.nn.sigmoid(gates[:, 0:h_dim])
            gf = jax.nn.sigmoid(gates[:, h_dim:2 * h_dim])
            gg = jnp.tanh(gates[:, 2 * h_dim:3 * h_dim])
            go = jax.nn.sigmoid(gates[:, 3 * h_dim:4 * h_dim])
            cc = gf * c_ref[...] + gi * gg
            c_ref[...] = cc
            h_ref[...] = go * jnp.tanh(cc)
            return 0

        lax.fori_loop(0, lseq, step, 0)
        out_ref[...] = (jnp.dot(h_ref[...], wfc_ref[...],
                                preferred_element_type=jnp.float32)
                        + bfc_ref[...])

    return pl.pallas_call(
        body,
        grid=(n_nodes // nb,),
        in_specs=[
            pl.BlockSpec((lseq, nb, xw), lambda i: (0, i, 0)),
            pl.BlockSpec((h_dim, 4 * h_dim), lambda i: (0, 0)),
            pl.BlockSpec((h_dim, 4 * h_dim), lambda i: (0, 0)),
            pl.BlockSpec((1, 4 * h_dim), lambda i: (0, 0)),
            pl.BlockSpec((h_dim, din), lambda i: (0, 0)),
            pl.BlockSpec((1, din), lambda i: (0, 0)),
        ],
        out_specs=pl.BlockSpec((nb, din), lambda i: (i, 0)),
        out_shape=jax.ShapeDtypeStruct((n_nodes, din), jnp.float32),
        scratch_shapes=[
            pltpu.VMEM((nb, h_dim), jnp.float32),
            pltpu.VMEM((nb, h_dim), jnp.float32),
        ],
    )(x3, wih_t, whh_t, b_row, wfc_t, bfc_row)


def _tc_rgcn_transform(hin, v_w, comp, alpha, nb):
    n_nodes, din = hin.shape
    b_dim, _, dout = v_w.shape
    r_dim = comp.shape[0]

    def body(hin_ref, v_ref, comp_ref, alpha_ref, out_ref):
        h = hin_ref[...]
        h = jnp.maximum(h, h * alpha_ref[...])
        w = jnp.dot(comp_ref[0],
                    v_ref[...].reshape(b_dim, din * dout),
                    preferred_element_type=jnp.float32).reshape(din, dout)
        out_ref[0] = jnp.dot(h, w, preferred_element_type=jnp.float32)

    return pl.pallas_call(
        body,
        grid=(n_nodes // nb, r_dim),
        in_specs=[
            pl.BlockSpec((nb, din), lambda i, r: (i, 0)),
            pl.BlockSpec((b_dim, din, dout), lambda i, r: (0, 0, 0)),
            pl.BlockSpec((1, 1, b_dim), lambda i, r: (r, 0, 0)),
            pl.BlockSpec((1, 1), lambda i, r: (0, 0)),
        ],
        out_specs=pl.BlockSpec((1, nb, dout), lambda i, r: (r, i, 0)),
        out_shape=jax.ShapeDtypeStruct((r_dim, n_nodes, dout), jnp.float32),
    )(hin, v_w, comp.reshape(r_dim, 1, b_dim), alpha.reshape(1, 1))


def _tc_combine(h, dout, nb):
    n_nodes, dpad = h.shape

    def body(p_ref, out_ref):
        out_ref[...] = p_ref[:, 0:dout]

    return pl.pallas_call(
        body,
        grid=(n_nodes // nb,),
        in_specs=[pl.BlockSpec((nb, dpad), lambda i: (i, 0))],
        out_specs=pl.BlockSpec((nb, dout), lambda i: (i, 0)),
        out_shape=jax.ShapeDtypeStruct((n_nodes, dout), jnp.float32),
    )(h)


def _pad_to(x, size):
    return jnp.pad(x, [(0, size - x.shape[0])] + [(0, 0)] * (x.ndim - 1))


def kernel(inputs, sequence_length, edge_index, rel_type, edge_norm, embed,
           W_ih_f, W_hh_f, b_ih_f, b_hh_f, W_ih_b, W_hh_b, b_ih_b, b_hh_b,
           W_fc, b_fc, V0, comp0, V1, comp1, V2, comp2):
    n_nodes, lseq = inputs.shape
    vocab, h_dim = embed.shape
    e_edges = edge_index.shape[1]
    r_dim = comp0.shape[0]
    din = V0.shape[1]
    dh = V1.shape[1]
    c_out = V2.shape[2]
    chunk = 128

    echunk = 80
    idx_flat = inputs.T.reshape(-1).astype(jnp.int32)
    n_per_w = -(-idx_flat.shape[0] // (NW * chunk)) * chunk
    idx3 = _pad_to(idx_flat, NW * n_per_w).reshape(NW, n_per_w // chunk, chunk)

    half = n_nodes // 2
    src = edge_index[0].astype(jnp.int32)
    dst = edge_index[1].astype(jnp.int32)
    rel = rel_type.astype(jnp.int32)
    eidx = rel * n_nodes + src
    norm = edge_norm.reshape(-1)
    flag = (dst >= half).astype(jnp.int32)
    sortperm = jnp.argsort(flag, stable=True)
    cnt1 = jnp.sum(flag)
    cnts = jnp.stack([e_edges - cnt1, cnt1])
    ecap_r = e_edges + NS * 2 * echunk
    jpos = jnp.arange(ecap_r)
    eidx_x = jnp.append(eidx, 0)
    dst_x = jnp.append(dst, 0)
    norm_x = jnp.append(norm, 0.0)
    eidx_rs, dst_rs, norm_rs = [], [], []
    for c_i in range(NC):
        start = 0 if c_i == 0 else cnts[0]
        valid = jpos < cnts[c_i]
        src_pos = jnp.minimum(start + jpos, e_edges - 1)
        take = jnp.where(valid, sortperm[src_pos], e_edges)
        eidx_rs.append(eidx_x[take])
        dst_rs.append(jnp.where(valid, dst_x[take] - c_i * half, half))
        norm_rs.append(norm_x[take])
    eidx_r = jnp.stack(eidx_rs).reshape(-1)
    dst_r = jnp.stack(dst_rs).reshape(-1)
    norm_r = jnp.stack(norm_rs).reshape(-1)
    svec = jnp.arange(NS)
    percw = (-(-cnts // (NS * echunk * 2))) * (echunk * 2)

    rem_c = jnp.clip(cnts[:, None] - svec[None, :] * percw[:, None],
                     0, percw[:, None])
    ncw = -(-rem_c // echunk)
    cnt2 = (((ncw + 1) // 2) * 2).astype(jnp.int32)
    base2 = (svec[None, :] * percw[:, None]).astype(jnp.int32)

    wih_t = W_ih_b.T
    whh_t = W_hh_b.T
    b_row = (b_ih_b + b_hh_b).reshape(1, 4 * h_dim)
    wfc_t = W_fc.T
    bfc_row = b_fc.reshape(1, din)

    embed_p = jnp.pad(embed, ((0, 0), (0, 128 - h_dim)))
    x_rows = _sc_embed_gather(embed_p, idx3, 128)
    x3 = x_rows[:lseq * n_nodes].reshape(lseq, n_nodes, 128)
    feats = _tc_lstm_fc(x3, wih_t, whh_t, b_row, wfc_t, bfc_row, nb=1000)

    v2p = jnp.pad(V2, ((0, 0), (0, 0), (0, dh - c_out)))
    v_stack = jnp.stack([V0, V1, v2p])
    comp_stack = jnp.stack([comp0, comp1, comp2])
    alpha_stack = jnp.array([1.0, 0.0, 0.0], jnp.float32)

    def layer_step(h, xs):
        v_w, comp, alpha = xs
        h_all = _tc_rgcn_transform(h, v_w, comp, alpha, nb=1000)
        h_new = _sc_edge_aggregate(h_all.reshape(r_dim * n_nodes, dh),
                                   eidx_r, dst_r, norm_r, cnt2, base2,
                                   half, dh, echunk, ecap_r)
        return h_new.reshape(n_nodes, dh), None

    h_fin, _ = lax.scan(layer_step, feats,
                        (v_stack, comp_stack, alpha_stack))
    return _tc_combine(h_fin, c_out, nb=1000)


# speedup vs baseline: 4.6166x; 4.6166x over previous
import functools

import jax
import jax.numpy as jnp
from jax import lax
from jax.experimental import pallas as pl
from jax.experimental.pallas import tpu as pltpu
from jax.experimental.pallas import tpu_sc as plsc

NC, NS, LANES = 2, 16, 16
NW = NC * NS


def _bcast_lane(v, j):
    idx = jnp.full((LANES, 1), j, jnp.int32)
    dnums = lax.GatherDimensionNumbers(
        offset_dims=(), collapsed_slice_dims=(0,), start_index_map=(0,))
    return lax.gather(v, idx, dnums, (1,),
                      mode=lax.GatherScatterMode.PROMISE_IN_BOUNDS)


def _mesh():
    return plsc.VectorSubcoreMesh(core_axis_name="c", subcore_axis_name="s",
                                  num_cores=NC, num_subcores=NS)


def _sc_embed_gather(table, idx3, D):
    nw, nchunks, chunk = idx3.shape
    total = nw * nchunks * chunk
    n_per_w = nchunks * chunk

    @functools.partial(
        pl.kernel,
        out_type=jax.ShapeDtypeStruct((total, D), jnp.float32),
        mesh=_mesh(),
        compiler_params=pltpu.CompilerParams(needs_layout_passes=False),
        scratch_types=[
            pltpu.VMEM((nchunks, chunk), jnp.int32),
            pltpu.VMEM((2, chunk, D), jnp.float32),
            pltpu.SemaphoreType.DMA,
        ],
    )
    def k(table_hbm, idx_hbm, out_hbm, idx_v, rows_v, gsem):
        wid = lax.axis_index("s") * NC + lax.axis_index("c")
        base = wid * n_per_w
        pltpu.sync_copy(idx_hbm.at[wid], idx_v)
        pltpu.async_copy(table_hbm.at[idx_v.at[0]], rows_v.at[0], gsem)

        def body(i, _):
            buf = lax.rem(i, 2)

            @pl.when(i + 1 < nchunks)
            def _start():
                pltpu.async_copy(table_hbm.at[idx_v.at[i + 1]],
                                 rows_v.at[1 - buf], gsem)

            pltpu.make_async_copy(table_hbm.at[idx_v.at[i]],
                                  rows_v.at[buf], gsem).wait()
            pltpu.sync_copy(
                rows_v.at[buf],
                out_hbm.at[pl.ds(pl.multiple_of(base + i * chunk, 8), chunk)])
            return 0

        lax.fori_loop(0, nchunks, body, 0)

    return k(table, idx3)


def _sc_edge_aggregate(h_all_flat, eidx3, dstr3, norm3, half, dout):
    ns_, nchunks, chunk = eidx3.shape
    acc_rows = half + 8
    zrows = -(-(-(-acc_rows // NS)) // 8) * 8
    erows = -(-(-(-half // NS)) // 8) * 8

    def _pieces(total):
        out_p, off = [], 0
        while off < total:
            sz = min(chunk, total - off)
            out_p.append((off, sz))
            off += sz
        return out_p

    jslices = dout // LANES
    groups = chunk // LANES

    @functools.partial(
        pl.kernel,
        out_type=jax.ShapeDtypeStruct((NC, half, dout), jnp.float32),
        mesh=_mesh(),
        compiler_params=pltpu.CompilerParams(needs_layout_passes=False),
        scratch_types=[
            pltpu.VMEM((4, chunk), jnp.int32),
            pltpu.VMEM((4, chunk), jnp.int32),
            pltpu.VMEM((4, chunk), jnp.float32),
            pltpu.VMEM((2, chunk, dout), jnp.float32),
            pltpu.VMEM((2, chunk, dout), jnp.float32),
            pltpu.VMEM_SHARED((acc_rows, dout), jnp.float32),
            pltpu.SemaphoreType.DMA,
            pltpu.SemaphoreType.DMA,
            pltpu.SemaphoreType.DMA,
        ],
    )
    def k(hall_hbm, eidx_hbm, dst_hbm, norm_hbm, out_hbm,
          eidx_m, dst_m, norm_m, rows_v, sbuf, acc, gsem, msem, ssem):
        c = lax.axis_index("c")
        s = lax.axis_index("s")
        w2 = c * NS + s

        def meta_start(i, slot):
            pltpu.async_copy(eidx_hbm.at[s, i], eidx_m.at[slot], msem)
            pltpu.async_copy(dst_hbm.at[w2, i], dst_m.at[slot], msem)
            pltpu.async_copy(norm_hbm.at[s, i], norm_m.at[slot], msem)

        def meta_wait(i, slot):
            pltpu.make_async_copy(eidx_hbm.at[s, i],
                                  eidx_m.at[slot], msem).wait()
            pltpu.make_async_copy(dst_hbm.at[w2, i],
                                  dst_m.at[slot], msem).wait()
            pltpu.make_async_copy(norm_hbm.at[s, i],
                                  norm_m.at[slot], msem).wait()

        zero16 = jnp.zeros((LANES,), jnp.float32)

        def zstore(r, _):
            for j in range(jslices):
                rows_v[0, r, pl.ds(j * LANES, LANES)] = zero16
            return 0

        lax.fori_loop(0, chunk, zstore, 0)
        zrow0 = pl.multiple_of(jnp.minimum(s * zrows, acc_rows - zrows), 8)
        for (zoff, zsz) in _pieces(zrows):
            pltpu.sync_copy(rows_v.at[0, pl.ds(0, zsz)],
                            acc.at[pl.ds(zrow0 + zoff, zsz)])
        plsc.subcore_barrier()

        meta_start(0, 0)
        meta_wait(0, 0)
        pltpu.async_copy(hall_hbm.at[eidx_m.at[0]], rows_v.at[0], gsem)

        @pl.when(1 < nchunks)
        def _prime():
            meta_start(1, 1)

        def chunk_step(i, buf):
            mb = lax.rem(i, 4)
            m1 = lax.rem(i + 1, 4)
            m2 = lax.rem(i + 2, 4)

            @pl.when(i + 1 < nchunks)
            def _start():
                meta_wait(i + 1, m1)
                pltpu.async_copy(hall_hbm.at[eidx_m.at[m1]],
                                 rows_v.at[1 - buf], gsem)

            @pl.when(i >= 2)
            def _drain():
                pltpu.make_async_copy(sbuf.at[buf],
                                      acc.at[dst_m.at[m2]], ssem).wait()

            @pl.when(i + 2 < nchunks)
            def _start2():
                meta_start(i + 2, m2)

            pltpu.make_async_copy(hall_hbm.at[eidx_m.at[mb]],
                                  rows_v.at[buf], gsem).wait()

            for g in range(groups):
                nv = norm_m[mb, pl.ds(g * LANES, LANES)]
                for j in range(LANES):
                    r = g * LANES + j
                    bj = _bcast_lane(nv, j)
                    for jj in range(jslices):
                        sl = pl.ds(jj * LANES, LANES)
                        sbuf[buf, r, sl] = rows_v[buf, r, sl] * bj

            pltpu.async_copy(sbuf.at[buf], acc.at[dst_m.at[mb]], ssem,
                             add=True)

        def pair(p, _):
            chunk_step(2 * p, 0)
            chunk_step(2 * p + 1, 1)
            return 0

        lax.fori_loop(0, nchunks // 2, pair, 0)
        pltpu.make_async_copy(sbuf.at[0],
                              acc.at[dst_m.at[0]], ssem).wait()
        pltpu.make_async_copy(sbuf.at[1],
                              acc.at[dst_m.at[0]], ssem).wait()
        plsc.subcore_barrier()
        erow0 = pl.multiple_of(jnp.minimum(s * erows, half - erows), 8)
        for (zoff, zsz) in _pieces(erows):
            pltpu.sync_copy(acc.at[pl.ds(erow0 + zoff, zsz)],
                            out_hbm.at[c, pl.ds(erow0 + zoff, zsz)])

    return k(h_all_flat, eidx3, dstr3, norm3)


def _tc_lstm_fc(x3, wih_t, whh_t, b_row, wfc_t, bfc_row, nb):
    lseq, n_nodes, xw = x3.shape
    h_dim = wih_t.shape[0]
    din = wfc_t.shape[1]

    def body(x_ref, wih_ref, whh_ref, b_ref, wfc_ref, bfc_ref, out_ref,
             h_ref, c_ref):
        h_ref[...] = jnp.zeros_like(h_ref)
        c_ref[...] = jnp.zeros_like(c_ref)

        def step(t, _):
            xt = x_ref[lseq - 1 - t][:, 0:h_dim]
            gates = (jnp.dot(xt, wih_ref[...],
                             preferred_element_type=jnp.float32)
                     + jnp.dot(h_ref[...], whh_ref[...],
                               preferred_element_type=jnp.float32)
                     + b_ref[...])
            gi = jax.nn.sigmoid(gates[:, 0:h_dim])
            gf = jax.nn.sigmoid(gates[:, h_dim:2 * h_dim])
            gg = jnp.tanh(gates[:, 2 * h_dim:3 * h_dim])
            go = jax.nn.sigmoid(gates[:, 3 * h_dim:4 * h_dim])
            cc = gf * c_ref[...] + gi * gg
            c_ref[...] = cc
            h_ref[...] = go * jnp.tanh(cc)
            return 0

        lax.fori_loop(0, lseq, step, 0)
        out_ref[...] = (jnp.dot(h_ref[...], wfc_ref[...],
                                preferred_element_type=jnp.float32)
                        + bfc_ref[...])

    return pl.pallas_call(
        body,
        grid=(n_nodes // nb,),
        in_specs=[
            pl.BlockSpec((lseq, nb, xw), lambda i: (0, i, 0)),
            pl.BlockSpec((h_dim, 4 * h_dim), lambda i: (0, 0)),
            pl.BlockSpec((h_dim, 4 * h_dim), lambda i: (0, 0)),
            pl.BlockSpec((1, 4 * h_dim), lambda i: (0, 0)),
            pl.BlockSpec((h_dim, din), lambda i: (0, 0)),
            pl.BlockSpec((1, din), lambda i: (0, 0)),
        ],
        out_specs=pl.BlockSpec((nb, din), lambda i: (i, 0)),
        out_shape=jax.ShapeDtypeStruct((n_nodes, din), jnp.float32),
        scratch_shapes=[
            pltpu.VMEM((nb, h_dim), jnp.float32),
            pltpu.VMEM((nb, h_dim), jnp.float32),
        ],
    )(x3, wih_t, whh_t, b_row, wfc_t, bfc_row)


def _tc_rgcn_transform(hin, v_w, comp, alpha, nb):
    n_nodes, din = hin.shape
    b_dim, _, dout = v_w.shape
    r_dim = comp.shape[0]

    def body(hin_ref, v_ref, comp_ref, alpha_ref, out_ref):
        h = hin_ref[...]
        h = jnp.maximum(h, h * alpha_ref[...])
        w = jnp.dot(comp_ref[0],
                    v_ref[...].reshape(b_dim, din * dout),
                    preferred_element_type=jnp.float32).reshape(din, dout)
        out_ref[0] = jnp.dot(h, w, preferred_element_type=jnp.float32)

    return pl.pallas_call(
        body,
        grid=(n_nodes // nb, r_dim),
        in_specs=[
            pl.BlockSpec((nb, din), lambda i, r: (i, 0)),
            pl.BlockSpec((b_dim, din, dout), lambda i, r: (0, 0, 0)),
            pl.BlockSpec((1, 1, b_dim), lambda i, r: (r, 0, 0)),
            pl.BlockSpec((1, 1), lambda i, r: (0, 0)),
        ],
        out_specs=pl.BlockSpec((1, nb, dout), lambda i, r: (r, i, 0)),
        out_shape=jax.ShapeDtypeStruct((r_dim, n_nodes, dout), jnp.float32),
    )(hin, v_w, comp.reshape(r_dim, 1, b_dim), alpha.reshape(1, 1))


def _tc_combine(h, dout, nb):
    n_nodes, dpad = h.shape

    def body(p_ref, out_ref):
        out_ref[...] = p_ref[:, 0:dout]

    return pl.pallas_call(
        body,
        grid=(n_nodes // nb,),
        in_specs=[pl.BlockSpec((nb, dpad), lambda i: (i, 0))],
        out_specs=pl.BlockSpec((nb, dout), lambda i: (i, 0)),
        out_shape=jax.ShapeDtypeStruct((n_nodes, dout), jnp.float32),
    )(h)


def _pad_to(x, size):
    return jnp.pad(x, [(0, size - x.shape[0])] + [(0, 0)] * (x.ndim - 1))


def kernel(inputs, sequence_length, edge_index, rel_type, edge_norm, embed,
           W_ih_f, W_hh_f, b_ih_f, b_hh_f, W_ih_b, W_hh_b, b_ih_b, b_hh_b,
           W_fc, b_fc, V0, comp0, V1, comp1, V2, comp2):
    n_nodes, lseq = inputs.shape
    vocab, h_dim = embed.shape
    e_edges = edge_index.shape[1]
    r_dim = comp0.shape[0]
    din = V0.shape[1]
    dh = V1.shape[1]
    c_out = V2.shape[2]
    chunk = 128

    echunk = 80
    idx_flat = inputs.T.reshape(-1).astype(jnp.int32)
    n_per_w = -(-idx_flat.shape[0] // (NW * chunk)) * chunk
    idx3 = _pad_to(idx_flat, NW * n_per_w).reshape(NW, n_per_w // chunk, chunk)

    half = n_nodes // 2
    src = edge_index[0].astype(jnp.int32)
    dst = edge_index[1].astype(jnp.int32)
    rel = rel_type.astype(jnp.int32)
    eidx = rel * n_nodes + src
    e_per_s = -(-e_edges // (NS * echunk * 2)) * (echunk * 2)
    e_tot = NS * e_per_s
    enchunks = e_per_s // echunk
    eidx3 = _pad_to(eidx, e_tot).reshape(NS, enchunks, echunk)
    dst_p = _pad_to(dst, e_tot)
    dstr3 = jnp.stack(
        [jnp.where(dst_p // half == c_i, dst_p - c_i * half, half)
         for c_i in range(NC)]).reshape(NC * NS, enchunks, echunk)
    norm3 = _pad_to(edge_norm.reshape(-1), e_tot).reshape(NS, enchunks, echunk)

    wih_t = W_ih_b.T
    whh_t = W_hh_b.T
    b_row = (b_ih_b + b_hh_b).reshape(1, 4 * h_dim)
    wfc_t = W_fc.T
    bfc_row = b_fc.reshape(1, din)

    embed_p = jnp.pad(embed, ((0, 0), (0, 128 - h_dim)))
    x_rows = _sc_embed_gather(embed_p, idx3, 128)
    x3 = x_rows[:lseq * n_nodes].reshape(lseq, n_nodes, 128)
    feats = _tc_lstm_fc(x3, wih_t, whh_t, b_row, wfc_t, bfc_row, nb=1000)

    v2p = jnp.pad(V2, ((0, 0), (0, 0), (0, dh - c_out)))
    v_stack = jnp.stack([V0, V1, v2p])
    comp_stack = jnp.stack([comp0, comp1, comp2])
    alpha_stack = jnp.array([1.0, 0.0, 0.0], jnp.float32)

    def layer_step(h, xs):
        v_w, comp, alpha = xs
        h_all = _tc_rgcn_transform(h, v_w, comp, alpha, nb=1000)
        h_new = _sc_edge_aggregate(h_all.reshape(r_dim * n_nodes, dh),
                                   eidx3, dstr3, norm3, half, dh)
        return h_new.reshape(n_nodes, dh), None

    h_fin, _ = lax.scan(layer_step, feats,
                        (v_stack, comp_stack, alpha_stack))
    return _tc_combine(h_fin, c_out, nb=1000)
